# baseline (speedup 1.0000x reference)
import functools

import jax
import jax.numpy as jnp
from jax import lax
from jax.experimental import pallas as pl
from jax.experimental.pallas import tpu as pltpu
from jax.experimental.pallas import tpu_sc as plsc

BATCH = 16384
FEATURE_DIM = 64
NUM_CLASSES = 100000
HIST = 100096
NC = 2
NS = 16
NW = NC * NS
PER_TILE = BATCH // NW
ROWS2D = BATCH // 128
GCHUNKS = PER_TILE // 128
HCHUNKS = (BATCH // NS) // 128
ZSLICE = HIST // NS

_mesh = plsc.VectorSubcoreMesh(core_axis_name="c", subcore_axis_name="s")


@functools.partial(
    pl.kernel,
    mesh=_mesh,
    out_type=(
        jax.ShapeDtypeStruct((BATCH, FEATURE_DIM), jnp.float32),
        jax.ShapeDtypeStruct((BATCH,), jnp.float32),
    ),
    scratch_types=(
        pltpu.VMEM((GCHUNKS, 128), jnp.int32),
        pltpu.VMEM((HCHUNKS, 128), jnp.int32),
        pltpu.VMEM((PER_TILE, FEATURE_DIM), jnp.float32),
        pltpu.VMEM((PER_TILE,), jnp.float32),
        pltpu.VMEM((128,), jnp.float32),
        pltpu.VMEM_SHARED((HIST,), jnp.float32),
        pltpu.SemaphoreType.DMA,
    ),
    compiler_params=pltpu.CompilerParams(use_tc_tiling_on_sc=False),
)
def _sc_gather_and_count(label2d_hbm, centers_hbm, zeros_hbm,
                         gath_hbm, cnt_hbm,
                         idx_my, idx_hist, rows_v, cnt_v, ones_v, hist_sh, sem):
    cid = lax.axis_index("c")
    sid = lax.axis_index("s")
    wid = sid * NC + cid

    pltpu.sync_copy(label2d_hbm.at[pl.ds(wid * GCHUNKS, GCHUNKS)], idx_my)
    pltpu.sync_copy(label2d_hbm.at[pl.ds(sid * HCHUNKS, HCHUNKS)], idx_hist)

    copies = [
        pltpu.async_copy(centers_hbm.at[idx_my.at[j]],
                         rows_v.at[pl.ds(j * 128, 128)], sem)
        for j in range(GCHUNKS)
    ]

    for k in range(128 // 16):
        ones_v[pl.ds(k * 16, 16)] = jnp.ones((16,), jnp.float32)

    pltpu.sync_copy(zeros_hbm.at[pl.ds(sid * ZSLICE, ZSLICE)],
                    hist_sh.at[pl.ds(sid * ZSLICE, ZSLICE)])
    plsc.subcore_barrier()
    for j in range(HCHUNKS):
        pltpu.sync_copy(ones_v, hist_sh.at[idx_hist.at[j]], add=True)
    plsc.subcore_barrier()

    for j in range(GCHUNKS):
        pltpu.sync_copy(hist_sh.at[idx_my.at[j]], cnt_v.at[pl.ds(j * 128, 128)])

    for cp in copies:
        cp.wait()
    base = wid * PER_TILE
    pltpu.sync_copy(rows_v, gath_hbm.at[pl.ds(base, PER_TILE)])
    pltpu.sync_copy(cnt_v, cnt_hbm.at[pl.ds(base, PER_TILE)])


def _tc_loss(f_ref, g_ref, c_ref, o_ref):
    d = f_ref[...] - g_ref[...]
    s = jnp.sum(d * d, axis=1)
    total = jnp.sum(jnp.sqrt(s / c_ref[...])) * (1.0 / BATCH)
    o_ref[...] = total.reshape(1, 1)


def kernel(feature, label, centers):
    label2d = label.reshape(ROWS2D, 128)
    zeros = jnp.zeros((HIST,), jnp.float32)
    gath, cnt = _sc_gather_and_count(label2d, centers, zeros)
    loss = pl.pallas_call(
        _tc_loss,
        out_shape=jax.ShapeDtypeStruct((1, 1), jnp.float32),
    )(feature, gath, cnt)
    return loss[0, 0]
